# baseline (speedup 1.0000x reference)
import jax
import jax.numpy as jnp
from jax.experimental import pallas as pl

_NUM_EXPERTS = 8
_TOP_K = 2
_BLOCK_ROWS = 256


def _moe_router_block(x_ref, w_ref, out_ref, gate_ref, idx_ref):
    x = x_ref[...]
    logits = jnp.dot(x, w_ref[...], preferred_element_type=jnp.float32)
    m = jnp.max(logits, axis=-1, keepdims=True)
    e = jnp.exp(logits - m)
    probs = e / jnp.sum(e, axis=-1, keepdims=True)
    iota = jax.lax.broadcasted_iota(jnp.int32, probs.shape, 1)
    g1 = jnp.max(probs, axis=-1, keepdims=True)
    i1 = jnp.min(jnp.where(probs == g1, iota, _NUM_EXPERTS), axis=-1,
                 keepdims=True)
    rest = jnp.where(iota == i1, -jnp.inf, probs)
    g2 = jnp.max(rest, axis=-1, keepdims=True)
    i2 = jnp.min(jnp.where(rest == g2, iota, _NUM_EXPERTS), axis=-1,
                 keepdims=True)
    gate_ref[...] = jnp.concatenate([g1, g2], axis=-1)
    idx_ref[...] = jnp.concatenate([i1, i2], axis=-1)
    out_ref[...] = x


def kernel(inputs, W):
    b, s, d = inputs.shape
    n_tokens = b * s
    x = inputs.reshape(n_tokens, d)
    grid = (n_tokens // _BLOCK_ROWS,)
    out, _, _ = pl.pallas_call(
        _moe_router_block,
        grid=grid,
        in_specs=[
            pl.BlockSpec((_BLOCK_ROWS, d), lambda i: (i, 0)),
            pl.BlockSpec((d, _NUM_EXPERTS), lambda i: (0, 0)),
        ],
        out_specs=[
            pl.BlockSpec((_BLOCK_ROWS, d), lambda i: (i, 0)),
            pl.BlockSpec((_BLOCK_ROWS, _TOP_K), lambda i: (i, 0)),
            pl.BlockSpec((_BLOCK_ROWS, _TOP_K), lambda i: (i, 0)),
        ],
        out_shape=[
            jax.ShapeDtypeStruct((n_tokens, d), jnp.float32),
            jax.ShapeDtypeStruct((n_tokens, _TOP_K), jnp.float32),
            jax.ShapeDtypeStruct((n_tokens, _TOP_K), jnp.int32),
        ],
    )(x, W)
    return out.reshape(inputs.shape)


# speedup vs baseline: 1.0564x; 1.0564x over previous
import jax
import jax.numpy as jnp
from jax.experimental import pallas as pl

_NUM_EXPERTS = 8
_TOP_K = 2
_BLOCK_ROWS = 512


def _moe_router_block(x_ref, w_ref, out_ref, gate_ref, idx_ref):
    x = x_ref[...]
    logits = jnp.dot(x, w_ref[...], preferred_element_type=jnp.float32)
    m = jnp.max(logits, axis=-1, keepdims=True)
    e = jnp.exp(logits - m)
    probs = e / jnp.sum(e, axis=-1, keepdims=True)
    iota = jax.lax.broadcasted_iota(jnp.int32, probs.shape, 1)
    g1 = jnp.max(probs, axis=-1, keepdims=True)
    i1 = jnp.min(jnp.where(probs == g1, iota, _NUM_EXPERTS), axis=-1,
                 keepdims=True)
    rest = jnp.where(iota == i1, -jnp.inf, probs)
    g2 = jnp.max(rest, axis=-1, keepdims=True)
    i2 = jnp.min(jnp.where(rest == g2, iota, _NUM_EXPERTS), axis=-1,
                 keepdims=True)
    gate_ref[...] = jnp.concatenate([g1, g2], axis=-1)
    idx_ref[...] = jnp.concatenate([i1, i2], axis=-1)
    out_ref[...] = x


def kernel(inputs, W):
    b, s, d = inputs.shape
    n_tokens = b * s
    x = inputs.reshape(n_tokens, d)
    grid = (n_tokens // _BLOCK_ROWS,)
    out, _, _ = pl.pallas_call(
        _moe_router_block,
        grid=grid,
        in_specs=[
            pl.BlockSpec((_BLOCK_ROWS, d), lambda i: (i, 0)),
            pl.BlockSpec((d, _NUM_EXPERTS), lambda i: (0, 0)),
        ],
        out_specs=[
            pl.BlockSpec((_BLOCK_ROWS, d), lambda i: (i, 0)),
            pl.BlockSpec((_BLOCK_ROWS, _TOP_K), lambda i: (i, 0)),
            pl.BlockSpec((_BLOCK_ROWS, _TOP_K), lambda i: (i, 0)),
        ],
        out_shape=[
            jax.ShapeDtypeStruct((n_tokens, d), jnp.float32),
            jax.ShapeDtypeStruct((n_tokens, _TOP_K), jnp.float32),
            jax.ShapeDtypeStruct((n_tokens, _TOP_K), jnp.int32),
        ],
    )(x, W)
    return out.reshape(inputs.shape)
